# Initial kernel scaffold; baseline (speedup 1.0000x reference)
import jax
import jax.numpy as jnp
from jax.experimental import pallas as pl


def kernel(x, edge_index, conv1_W, conv1_b, conv2_W, conv2_b, ln1_W, ln1_b, ln2_W, ln2_b, a1_W, a1_b, a2_W, a2_b, a3_W, a3_b, a4_W, a4_b, f1_W, f1_b, f2_W, f2_b, f3_W, f3_b):
    raise NotImplementedError("write your pallas kernel here")


# speedup vs baseline: 49.2220x; 49.2220x over previous
import functools

import jax
import jax.numpy as jnp
from jax import lax
from jax.experimental import pallas as pl
from jax.experimental.pallas import tpu as pltpu
from jax.experimental.pallas import tpu_sc as plsc

N = 100000
F = 16
E = 3200000

NC = 2
NS = 16
NW = NC * NS

LANE = 128
JROWS = 16
CHUNK = JROWS * LANE
CH_PER_W = -(-E // (NW * CHUNK))
E_PAD = NW * CH_PER_W * CHUNK
IDX_ROWS = E_PAD // LANE
ROWS_PER_W = IDX_ROWS // NW

SJ = 8
SCHUNK = SJ * LANE
SCH_PER_W = E_PAD // (NW * SCHUNK)

NP = 100096
SLICE = NP // NS
ZR = SCHUNK

_sc_mesh = plsc.VectorSubcoreMesh(
    core_axis_name="c", subcore_axis_name="s", num_cores=NC, num_subcores=NS)


def _worker_chunk_base(c, s, g, jrows):
  wid = c * NS + s
  return wid * ROWS_PER_W + g * jrows


@functools.partial(
    pl.kernel,
    out_type=jax.ShapeDtypeStruct((NC * NP,), jnp.float32),
    mesh=_sc_mesh,
    scratch_types=[
        pltpu.VMEM_SHARED((NP,), jnp.float32),
        pltpu.VMEM((JROWS, LANE), jnp.int32),
        pltpu.VMEM((LANE,), jnp.float32),
        pltpu.VMEM((SLICE,), jnp.float32),
    ],
)
def _sc_degree(dst_hbm, deg_out, deg_sh, didx, ones_v, bounce):
  c = lax.axis_index("c")
  s = lax.axis_index("s")

  for i in range(LANE // 16):
    ones_v[pl.ds(i * 16, 16)] = jnp.ones((16,), jnp.float32)

  def zero_body(i, _):
    bounce[pl.ds(i * 16, 16)] = jnp.zeros((16,), jnp.float32)
    return _
  lax.fori_loop(0, SLICE // 16, zero_body, None)
  pltpu.sync_copy(bounce, deg_sh.at[pl.ds(s * SLICE, SLICE)])
  plsc.subcore_barrier()

  def chunk_body(g, _):
    rb = _worker_chunk_base(c, s, g, JROWS)
    pltpu.sync_copy(dst_hbm.at[pl.ds(rb, JROWS)], didx)
    for j in range(JROWS):
      pltpu.sync_copy(ones_v, deg_sh.at[didx.at[j]], add=True)
    return _
  lax.fori_loop(0, CH_PER_W, chunk_body, None)

  plsc.subcore_barrier()
  pltpu.sync_copy(deg_sh.at[pl.ds(s * SLICE, SLICE)], bounce)
  pltpu.sync_copy(bounce, deg_out.at[pl.ds(c * NP + s * SLICE, SLICE)])


@functools.partial(
    pl.kernel,
    out_type=jax.ShapeDtypeStruct((NC, NP, F), jnp.float32),
    mesh=_sc_mesh,
    compiler_params=pltpu.CompilerParams(use_tc_tiling_on_sc=False),
    scratch_types=[
        pltpu.VMEM_SHARED((NP, F), jnp.float32),
        pltpu.VMEM((SJ, LANE), jnp.int32),
        pltpu.VMEM((SJ, LANE), jnp.int32),
        pltpu.VMEM((SCHUNK, F), jnp.float32),
        pltpu.SemaphoreType.DMA,
        pltpu.SemaphoreType.DMA,
    ],
)
def _sc_segsum(y_hbm, src_hbm, dst_hbm, acc_out,
               acc_sh, sidx, didx, rows, gsem, ssem):
  c = lax.axis_index("c")
  s = lax.axis_index("s")

  def zero_body(i, _):
    rows[i, :] = jnp.zeros((F,), jnp.float32)
    return _
  lax.fori_loop(0, ZR, zero_body, None)

  base = s * SLICE
  nfull = SLICE // ZR
  rem = SLICE - nfull * ZR
  for k in range(nfull):
    pltpu.sync_copy(rows, acc_sh.at[pl.ds(base + k * ZR, ZR)])
  if rem:
    pltpu.sync_copy(rows.at[pl.ds(0, rem)],
                    acc_sh.at[pl.ds(base + nfull * ZR, rem)])
  plsc.subcore_barrier()

  def chunk_body(g, _):
    rb = _worker_chunk_base(c, s, g, SJ)
    pltpu.sync_copy(src_hbm.at[pl.ds(rb, SJ)], sidx)
    pltpu.sync_copy(dst_hbm.at[pl.ds(rb, SJ)], didx)
    gathers = [
        pltpu.async_copy(y_hbm.at[sidx.at[j]],
                         rows.at[pl.ds(j * LANE, LANE)], gsem)
        for j in range(SJ)
    ]
    for d in gathers:
      d.wait()
    scatters = [
        pltpu.async_copy(rows.at[pl.ds(j * LANE, LANE)],
                         acc_sh.at[didx.at[j]], ssem, add=True)
        for j in range(SJ)
    ]
    for d in scatters:
      d.wait()
    return _
  lax.fori_loop(0, SCH_PER_W, chunk_body, None)

  plsc.subcore_barrier()
  for k in range(nfull):
    pltpu.sync_copy(acc_sh.at[pl.ds(base + k * ZR, ZR)], rows)
    pltpu.sync_copy(rows, acc_out.at[c, pl.ds(base + k * ZR, ZR)])
  if rem:
    pltpu.sync_copy(acc_sh.at[pl.ds(base + nfull * ZR, rem)],
                    rows.at[pl.ds(0, rem)])
    pltpu.sync_copy(rows.at[pl.ds(0, rem)],
                    acc_out.at[c, pl.ds(base + nfull * ZR, rem)])


R = 2000
GRID = N // R


def _row_spec(cols):
  return pl.BlockSpec((R, cols), lambda i: (i, 0))


def _full_spec(shape):
  nd = len(shape)
  return pl.BlockSpec(shape, lambda i: (0,) * nd)


def _tc_stage1_body(x_ref, d0_ref, d1_ref, w1_ref, l1w_ref, l1b_ref,
                    l2w_ref, l2b_ref, dinv_ref, y1_ref, gg2_ref):
  deg = d0_ref[...] + d1_ref[...] + 1.0
  dinv = lax.rsqrt(deg)
  dinv_ref[...] = dinv
  x = x_ref[...]
  xw = jnp.dot(x, w1_ref[...], preferred_element_type=jnp.float32)
  y1_ref[...] = xw * dinv
  g1 = jnp.maximum(
      jnp.dot(x, l1w_ref[...], preferred_element_type=jnp.float32)
      + l1b_ref[...], 0.0)
  gg2_ref[...] = jnp.maximum(
      jnp.dot(g1, l2w_ref[...], preferred_element_type=jnp.float32)
      + l2b_ref[...], 0.0)


def _tc_stage2_body(a0_ref, a1_ref, y1_ref, dinv_ref, b1_ref, w2_ref,
                    a1w_ref, a1b_ref, a2w_ref, a2b_ref, y2_ref, xa1_ref):
  dinv = dinv_ref[...]
  x1 = jnp.maximum(
      dinv * (a0_ref[...] + a1_ref[...] + y1_ref[...]) + b1_ref[...], 0.0)
  y2_ref[...] = jnp.dot(
      x1, w2_ref[...], preferred_element_type=jnp.float32) * dinv
  t = jnp.maximum(
      jnp.dot(x1, a1w_ref[...], preferred_element_type=jnp.float32)
      + a1b_ref[...], 0.0)
  xa1_ref[...] = jnp.maximum(
      jnp.dot(t, a2w_ref[...], preferred_element_type=jnp.float32)
      + a2b_ref[...], 0.0)


def _tc_stage3_body(a0_ref, a1_ref, y2_ref, dinv_ref, b2_ref, gg2_ref,
                    xa1_ref, a3w_ref, a3b_ref, a4w_ref, a4b_ref,
                    f1a_ref, f1b_w_ref, f1c_ref, f1b_ref,
                    f2w_ref, f2b_ref, f3w_ref, f3b_ref, out_ref):
  dinv = dinv_ref[...]
  x2 = jnp.maximum(
      dinv * (a0_ref[...] + a1_ref[...] + y2_ref[...]) + b2_ref[...], 0.0)
  t = jnp.maximum(
      jnp.dot(x2, a3w_ref[...], preferred_element_type=jnp.float32)
      + a3b_ref[...], 0.0)
  xa2 = jnp.maximum(
      jnp.dot(t, a4w_ref[...], preferred_element_type=jnp.float32)
      + a4b_ref[...], 0.0)
  f = (jnp.dot(gg2_ref[...], f1a_ref[...], preferred_element_type=jnp.float32)
       + jnp.dot(xa1_ref[...], f1b_w_ref[...],
                 preferred_element_type=jnp.float32)
       + jnp.dot(xa2, f1c_ref[...], preferred_element_type=jnp.float32)
       + f1b_ref[...])
  f = jnp.maximum(f, 0.0)
  f = jnp.maximum(
      jnp.dot(f, f2w_ref[...], preferred_element_type=jnp.float32)
      + f2b_ref[...], 0.0)
  o = jnp.dot(f, f3w_ref[...], preferred_element_type=jnp.float32) + f3b_ref[...]
  out_ref[...] = jax.nn.sigmoid(o)


def kernel(x, edge_index, conv1_W, conv1_b, conv2_W, conv2_b,
           ln1_W, ln1_b, ln2_W, ln2_b,
           a1_W, a1_b, a2_W, a2_b, a3_W, a3_b, a4_W, a4_b,
           f1_W, f1_b, f2_W, f2_b, f3_W, f3_b):
  pad = E_PAD - E
  pad_idx = N + (jnp.arange(pad, dtype=jnp.int32) % 64)
  src2d = jnp.concatenate([edge_index[0], pad_idx]).reshape(IDX_ROWS, LANE)
  dst2d = jnp.concatenate([edge_index[1], pad_idx]).reshape(IDX_ROWS, LANE)

  degp = _sc_degree(dst2d).reshape(NC, NP)
  deg0 = degp[0, :N].reshape(N, 1)
  deg1 = degp[1, :N].reshape(N, 1)

  tc1 = pl.pallas_call(
      _tc_stage1_body,
      grid=(GRID,),
      in_specs=[
          _row_spec(F), _row_spec(1), _row_spec(1),
          _full_spec((F, F)), _full_spec((16, 32)), _full_spec((1, 32)),
          _full_spec((32, 16)), _full_spec((1, 16)),
      ],
      out_specs=[_row_spec(1), _row_spec(F), _row_spec(F)],
      out_shape=[
          jax.ShapeDtypeStruct((N, 1), jnp.float32),
          jax.ShapeDtypeStruct((N, F), jnp.float32),
          jax.ShapeDtypeStruct((N, F), jnp.float32),
      ],
  )
  dinv, y1, gg2 = tc1(x, deg0, deg1, conv1_W, ln1_W, ln1_b.reshape(1, 32),
                      ln2_W, ln2_b.reshape(1, 16))

  zpad = jnp.zeros((NP - N, F), jnp.float32)
  acc1 = _sc_segsum(jnp.concatenate([y1, zpad]), src2d, dst2d)

  tc2 = pl.pallas_call(
      _tc_stage2_body,
      grid=(GRID,),
      in_specs=[
          _row_spec(F), _row_spec(F), _row_spec(F), _row_spec(1),
          _full_spec((1, F)), _full_spec((F, F)),
          _full_spec((F, 16)), _full_spec((1, 16)),
          _full_spec((16, 16)), _full_spec((1, 16)),
      ],
      out_specs=[_row_spec(F), _row_spec(F)],
      out_shape=[
          jax.ShapeDtypeStruct((N, F), jnp.float32),
          jax.ShapeDtypeStruct((N, F), jnp.float32),
      ],
  )
  y2, xa1 = tc2(acc1[0, :N], acc1[1, :N], y1, dinv, conv1_b.reshape(1, F),
                conv2_W, a1_W, a1_b.reshape(1, 16), a2_W, a2_b.reshape(1, 16))

  acc2 = _sc_segsum(jnp.concatenate([y2, zpad]), src2d, dst2d)

  tc3 = pl.pallas_call(
      _tc_stage3_body,
      grid=(GRID,),
      in_specs=[
          _row_spec(F), _row_spec(F), _row_spec(F), _row_spec(1),
          _full_spec((1, F)), _row_spec(F), _row_spec(F),
          _full_spec((F, 16)), _full_spec((1, 16)),
          _full_spec((16, 16)), _full_spec((1, 16)),
          _full_spec((16, 64)), _full_spec((16, 64)), _full_spec((16, 64)),
          _full_spec((1, 64)),
          _full_spec((64, 32)), _full_spec((1, 32)),
          _full_spec((32, 1)), _full_spec((1, 1)),
      ],
      out_specs=[_row_spec(1)],
      out_shape=[jax.ShapeDtypeStruct((N, 1), jnp.float32)],
  )
  (out,) = tc3(acc2[0, :N], acc2[1, :N], y2, dinv, conv2_b.reshape(1, F),
               gg2, xa1, a3_W, a3_b.reshape(1, 16), a4_W, a4_b.reshape(1, 16),
               f1_W[:16], f1_W[16:32], f1_W[32:48], f1_b.reshape(1, 64),
               f2_W, f2_b.reshape(1, 32), f3_W, f3_b.reshape(1, 1))
  return out


# speedup vs baseline: 57.3118x; 1.1644x over previous
import functools

import jax
import jax.numpy as jnp
from jax import lax
from jax.experimental import pallas as pl
from jax.experimental.pallas import tpu as pltpu
from jax.experimental.pallas import tpu_sc as plsc

N = 100000
F = 16
E = 3200000

NC = 2
NS = 16
NW = NC * NS

LANE = 128
JROWS = 16
CHUNK = JROWS * LANE
CH_PER_W = -(-E // (NW * CHUNK))
E_PAD = NW * CH_PER_W * CHUNK
IDX_ROWS = E_PAD // LANE
ROWS_PER_W = IDX_ROWS // NW

SJ = 8
SCHUNK = SJ * LANE
SCH_PER_W = E_PAD // (NW * SCHUNK)

NP = 100352
SLICE = NP // NS
ZR = SCHUNK

_sc_mesh = plsc.VectorSubcoreMesh(
    core_axis_name="c", subcore_axis_name="s", num_cores=NC, num_subcores=NS)


def _worker_chunk_base(c, s, g, jrows):
  wid = c * NS + s
  return wid * ROWS_PER_W + g * jrows


@functools.partial(
    pl.kernel,
    out_type=jax.ShapeDtypeStruct((NC * NP,), jnp.float32),
    mesh=_sc_mesh,
    scratch_types=[
        pltpu.VMEM_SHARED((NP,), jnp.float32),
        pltpu.VMEM((JROWS, LANE), jnp.int32),
        pltpu.VMEM((LANE,), jnp.float32),
        pltpu.VMEM((SLICE,), jnp.float32),
    ],
)
def _sc_degree(dst_hbm, deg_out, deg_sh, didx, ones_v, bounce):
  c = lax.axis_index("c")
  s = lax.axis_index("s")

  for i in range(LANE // 16):
    ones_v[pl.ds(i * 16, 16)] = jnp.ones((16,), jnp.float32)

  def zero_body(i, _):
    bounce[pl.ds(i * 16, 16)] = jnp.zeros((16,), jnp.float32)
    return _
  lax.fori_loop(0, SLICE // 16, zero_body, None)
  pltpu.sync_copy(bounce, deg_sh.at[pl.ds(s * SLICE, SLICE)])
  plsc.subcore_barrier()

  def chunk_body(g, _):
    rb = _worker_chunk_base(c, s, g, JROWS)
    pltpu.sync_copy(dst_hbm.at[pl.ds(rb, JROWS)], didx)
    for j in range(JROWS):
      pltpu.sync_copy(ones_v, deg_sh.at[didx.at[j]], add=True)
    return _
  lax.fori_loop(0, CH_PER_W, chunk_body, None)

  plsc.subcore_barrier()
  pltpu.sync_copy(deg_sh.at[pl.ds(s * SLICE, SLICE)], bounce)
  pltpu.sync_copy(bounce, deg_out.at[pl.ds(c * NP + s * SLICE, SLICE)])


@functools.partial(
    pl.kernel,
    out_type=jax.ShapeDtypeStruct((NC, NP, F), jnp.float32),
    mesh=_sc_mesh,
    compiler_params=pltpu.CompilerParams(use_tc_tiling_on_sc=False),
    scratch_types=[
        pltpu.VMEM_SHARED((NP, F), jnp.float32),
        pltpu.VMEM((SJ, LANE), jnp.int32),
        pltpu.VMEM((SJ, LANE), jnp.int32),
        pltpu.VMEM((SCHUNK, F), jnp.float32),
        pltpu.SemaphoreType.DMA,
        pltpu.SemaphoreType.DMA,
    ],
)
def _sc_segsum(y_hbm, src_hbm, dst_hbm, acc_out,
               acc_sh, sidx, didx, rows, gsem, ssem):
  c = lax.axis_index("c")
  s = lax.axis_index("s")

  def zero_body(i, _):
    rows[i, :] = jnp.zeros((F,), jnp.float32)
    return _
  lax.fori_loop(0, ZR, zero_body, None)

  base = s * SLICE
  nfull = SLICE // ZR
  rem = SLICE - nfull * ZR
  for k in range(nfull):
    pltpu.sync_copy(rows, acc_sh.at[pl.ds(base + k * ZR, ZR)])
  if rem:
    pltpu.sync_copy(rows.at[pl.ds(0, rem)],
                    acc_sh.at[pl.ds(base + nfull * ZR, rem)])
  plsc.subcore_barrier()

  def chunk_body(g, _):
    rb = _worker_chunk_base(c, s, g, SJ)
    pltpu.sync_copy(src_hbm.at[pl.ds(rb, SJ)], sidx)
    pltpu.sync_copy(dst_hbm.at[pl.ds(rb, SJ)], didx)
    gathers = [
        pltpu.async_copy(y_hbm.at[sidx.at[j]],
                         rows.at[pl.ds(j * LANE, LANE)], gsem)
        for j in range(SJ)
    ]
    for d in gathers:
      d.wait()
    scatters = [
        pltpu.async_copy(rows.at[pl.ds(j * LANE, LANE)],
                         acc_sh.at[didx.at[j]], ssem, add=True)
        for j in range(SJ)
    ]
    for d in scatters:
      d.wait()
    return _
  lax.fori_loop(0, SCH_PER_W, chunk_body, None)

  plsc.subcore_barrier()
  for k in range(nfull):
    pltpu.sync_copy(acc_sh.at[pl.ds(base + k * ZR, ZR)], rows)
    pltpu.sync_copy(rows, acc_out.at[c, pl.ds(base + k * ZR, ZR)])
  if rem:
    pltpu.sync_copy(acc_sh.at[pl.ds(base + nfull * ZR, rem)],
                    rows.at[pl.ds(0, rem)])
    pltpu.sync_copy(rows.at[pl.ds(0, rem)],
                    acc_out.at[c, pl.ds(base + nfull * ZR, rem)])


R = 3584
GRID = NP // R


def _row_spec(cols):
  return pl.BlockSpec((R, cols), lambda i: (i, 0))


def _full_spec(shape):
  nd = len(shape)
  return pl.BlockSpec(shape, lambda i: (0,) * nd)


def _tc_stage1_body(x_ref, deg_ref, w1_ref, l1w_ref, l1b_ref,
                    l2w_ref, l2b_ref, dinv_ref, y1_ref, gg2_ref):
  d = deg_ref[...]
  deg_row = d[0:1, :] + d[1:2, :] + 1.0
  dinv = jnp.transpose(lax.rsqrt(deg_row)) * jnp.ones((1, F), jnp.float32)
  dinv_ref[...] = dinv
  x = x_ref[...]
  xw = jnp.dot(x, w1_ref[...], preferred_element_type=jnp.float32)
  y1_ref[...] = xw * dinv
  g1 = jnp.maximum(
      jnp.dot(x, l1w_ref[...], preferred_element_type=jnp.float32)
      + l1b_ref[...], 0.0)
  gg2_ref[...] = jnp.maximum(
      jnp.dot(g1, l2w_ref[...], preferred_element_type=jnp.float32)
      + l2b_ref[...], 0.0)


def _tc_stage2_body(acc_ref, y1_ref, dinv_ref, b1_ref, w2_ref,
                    a1w_ref, a1b_ref, a2w_ref, a2b_ref, y2_ref, xa1_ref):
  dinv = dinv_ref[...]
  x1 = jnp.maximum(
      dinv * (acc_ref[0] + acc_ref[1] + y1_ref[...]) + b1_ref[...], 0.0)
  y2_ref[...] = jnp.dot(
      x1, w2_ref[...], preferred_element_type=jnp.float32) * dinv
  t = jnp.maximum(
      jnp.dot(x1, a1w_ref[...], preferred_element_type=jnp.float32)
      + a1b_ref[...], 0.0)
  xa1_ref[...] = jnp.maximum(
      jnp.dot(t, a2w_ref[...], preferred_element_type=jnp.float32)
      + a2b_ref[...], 0.0)


def _tc_stage3_body(acc_ref, y2_ref, dinv_ref, b2_ref, gg2_ref,
                    xa1_ref, a3w_ref, a3b_ref, a4w_ref, a4b_ref,
                    f1a_ref, f1b_w_ref, f1c_ref, f1b_ref,
                    f2w_ref, f2b_ref, f3w_ref, f3b_ref, out_ref):
  dinv = dinv_ref[...]
  x2 = jnp.maximum(
      dinv * (acc_ref[0] + acc_ref[1] + y2_ref[...]) + b2_ref[...], 0.0)
  t = jnp.maximum(
      jnp.dot(x2, a3w_ref[...], preferred_element_type=jnp.float32)
      + a3b_ref[...], 0.0)
  xa2 = jnp.maximum(
      jnp.dot(t, a4w_ref[...], preferred_element_type=jnp.float32)
      + a4b_ref[...], 0.0)
  f = (jnp.dot(gg2_ref[...], f1a_ref[...], preferred_element_type=jnp.float32)
       + jnp.dot(xa1_ref[...], f1b_w_ref[...],
                 preferred_element_type=jnp.float32)
       + jnp.dot(xa2, f1c_ref[...], preferred_element_type=jnp.float32)
       + f1b_ref[...])
  f = jnp.maximum(f, 0.0)
  f = jnp.maximum(
      jnp.dot(f, f2w_ref[...], preferred_element_type=jnp.float32)
      + f2b_ref[...], 0.0)
  o = jnp.dot(f, f3w_ref[...], preferred_element_type=jnp.float32) + f3b_ref[...]
  out_ref[...] = jax.nn.sigmoid(o)


def kernel(x, edge_index, conv1_W, conv1_b, conv2_W, conv2_b,
           ln1_W, ln1_b, ln2_W, ln2_b,
           a1_W, a1_b, a2_W, a2_b, a3_W, a3_b, a4_W, a4_b,
           f1_W, f1_b, f2_W, f2_b, f3_W, f3_b):
  pad = E_PAD - E
  pad_idx = N + (jnp.arange(pad, dtype=jnp.int32) % 64)
  src2d = jnp.concatenate([edge_index[0], pad_idx]).reshape(IDX_ROWS, LANE)
  dst2d = jnp.concatenate([edge_index[1], pad_idx]).reshape(IDX_ROWS, LANE)

  xp = jnp.concatenate([x, jnp.zeros((NP - N, F), jnp.float32)])
  degp = _sc_degree(dst2d).reshape(NC, NP)

  deg_spec = pl.BlockSpec((NC, R), lambda i: (0, i))
  acc_spec = pl.BlockSpec((NC, R, F), lambda i: (0, i, 0))

  tc1 = pl.pallas_call(
      _tc_stage1_body,
      grid=(GRID,),
      in_specs=[
          _row_spec(F), deg_spec,
          _full_spec((F, F)), _full_spec((16, 32)), _full_spec((1, 32)),
          _full_spec((32, 16)), _full_spec((1, 16)),
      ],
      out_specs=[_row_spec(F), _row_spec(F), _row_spec(F)],
      out_shape=[
          jax.ShapeDtypeStruct((NP, F), jnp.float32),
          jax.ShapeDtypeStruct((NP, F), jnp.float32),
          jax.ShapeDtypeStruct((NP, F), jnp.float32),
      ],
  )
  dinv, y1, gg2 = tc1(xp, degp, conv1_W, ln1_W, ln1_b.reshape(1, 32),
                      ln2_W, ln2_b.reshape(1, 16))

  acc1 = _sc_segsum(y1, src2d, dst2d)

  tc2 = pl.pallas_call(
      _tc_stage2_body,
      grid=(GRID,),
      in_specs=[
          acc_spec, _row_spec(F), _row_spec(F),
          _full_spec((1, F)), _full_spec((F, F)),
          _full_spec((F, 16)), _full_spec((1, 16)),
          _full_spec((16, 16)), _full_spec((1, 16)),
      ],
      out_specs=[_row_spec(F), _row_spec(F)],
      out_shape=[
          jax.ShapeDtypeStruct((NP, F), jnp.float32),
          jax.ShapeDtypeStruct((NP, F), jnp.float32),
      ],
  )
  y2, xa1 = tc2(acc1, y1, dinv, conv1_b.reshape(1, F),
                conv2_W, a1_W, a1_b.reshape(1, 16), a2_W, a2_b.reshape(1, 16))

  acc2 = _sc_segsum(y2, src2d, dst2d)

  tc3 = pl.pallas_call(
      _tc_stage3_body,
      grid=(GRID,),
      in_specs=[
          acc_spec, _row_spec(F), _row_spec(F),
          _full_spec((1, F)), _row_spec(F), _row_spec(F),
          _full_spec((F, 16)), _full_spec((1, 16)),
          _full_spec((16, 16)), _full_spec((1, 16)),
          _full_spec((16, 64)), _full_spec((16, 64)), _full_spec((16, 64)),
          _full_spec((1, 64)),
          _full_spec((64, 32)), _full_spec((1, 32)),
          _full_spec((32, 1)), _full_spec((1, 1)),
      ],
      out_specs=[_row_spec(1)],
      out_shape=[jax.ShapeDtypeStruct((NP, 1), jnp.float32)],
  )
  (out,) = tc3(acc2, y2, dinv, conv2_b.reshape(1, F),
               gg2, xa1, a3_W, a3_b.reshape(1, 16), a4_W, a4_b.reshape(1, 16),
               f1_W[:16], f1_W[16:32], f1_W[32:48], f1_b.reshape(1, 64),
               f2_W, f2_b.reshape(1, 32), f3_W, f3_b.reshape(1, 1))
  return out[:N]


# speedup vs baseline: 65.5088x; 1.1430x over previous
import functools

import jax
import jax.numpy as jnp
from jax import lax
from jax.experimental import pallas as pl
from jax.experimental.pallas import tpu as pltpu
from jax.experimental.pallas import tpu_sc as plsc

N = 100000
F = 16
E = 3200000

NC = 2
NS = 16
NW = NC * NS

LANE = 128
ROWS_PER_W = 792
IDX_ROWS = NW * ROWS_PER_W
E_PAD = IDX_ROWS * LANE

DJ = 12
DCH = ROWS_PER_W // DJ

SJ = 4
SCHUNK = SJ * LANE
SCH = ROWS_PER_W // SJ

NP = 100352
SLICE = NP // NS
ZR = 1024

_sc_mesh = plsc.VectorSubcoreMesh(
    core_axis_name="c", subcore_axis_name="s", num_cores=NC, num_subcores=NS)


def _worker_chunk_base(c, s, g, jrows):
  wid = c * NS + s
  return wid * ROWS_PER_W + g * jrows


@functools.partial(
    pl.kernel,
    out_type=jax.ShapeDtypeStruct((NC * NP,), jnp.float32),
    mesh=_sc_mesh,
    compiler_params=pltpu.CompilerParams(use_tc_tiling_on_sc=False),
    scratch_types=[
        pltpu.VMEM_SHARED((NP,), jnp.float32),
        pltpu.VMEM((2, DJ, LANE), jnp.int32),
        pltpu.VMEM((LANE,), jnp.float32),
        pltpu.VMEM((SLICE,), jnp.float32),
        pltpu.SemaphoreType.DMA,
        pltpu.SemaphoreType.DMA,
    ],
)
def _sc_degree(dst_hbm, deg_out, deg_sh, didx, ones_v, bounce, sem0, sem1):
  c = lax.axis_index("c")
  s = lax.axis_index("s")
  sems = (sem0, sem1)

  for i in range(LANE // 16):
    ones_v[pl.ds(i * 16, 16)] = jnp.ones((16,), jnp.float32)

  def zero_body(i, _):
    bounce[pl.ds(i * 16, 16)] = jnp.zeros((16,), jnp.float32)
    return _
  lax.fori_loop(0, SLICE // 16, zero_body, None)
  pltpu.sync_copy(bounce, deg_sh.at[pl.ds(s * SLICE, SLICE)])
  plsc.subcore_barrier()

  def chunk_pair(i, _):
    for b in range(2):
      g = 2 * i + b

      @pl.when(i > 0)
      def _drain():
        for j in range(DJ):
          pltpu.make_async_copy(
              ones_v, deg_sh.at[didx.at[b].at[j]], sems[b]).wait()

      rb = _worker_chunk_base(c, s, g, DJ)
      pltpu.sync_copy(dst_hbm.at[pl.ds(rb, DJ)], didx.at[b])
      for j in range(DJ):
        pltpu.async_copy(ones_v, deg_sh.at[didx.at[b].at[j]], sems[b],
                         add=True)
    return _
  lax.fori_loop(0, DCH // 2, chunk_pair, None)
  for b in range(2):
    for j in range(DJ):
      pltpu.make_async_copy(
          ones_v, deg_sh.at[didx.at[b].at[j]], sems[b]).wait()

  plsc.subcore_barrier()
  pltpu.sync_copy(deg_sh.at[pl.ds(s * SLICE, SLICE)], bounce)
  pltpu.sync_copy(bounce, deg_out.at[pl.ds(c * NP + s * SLICE, SLICE)])


@functools.partial(
    pl.kernel,
    out_type=jax.ShapeDtypeStruct((NC, NP, F), jnp.float32),
    mesh=_sc_mesh,
    compiler_params=pltpu.CompilerParams(use_tc_tiling_on_sc=False),
    scratch_types=[
        pltpu.VMEM_SHARED((NP, F), jnp.float32),
        pltpu.VMEM((3, SJ, LANE), jnp.int32),
        pltpu.VMEM((3, SJ, LANE), jnp.int32),
        pltpu.VMEM((3, SCHUNK, F), jnp.float32),
        pltpu.SemaphoreType.DMA,
        pltpu.SemaphoreType.DMA,
        pltpu.SemaphoreType.DMA,
        pltpu.SemaphoreType.DMA,
        pltpu.SemaphoreType.DMA,
        pltpu.SemaphoreType.DMA,
    ],
)
def _sc_segsum(y_hbm, src_hbm, dst_hbm, acc_out,
               acc_sh, sidx, didx, rows,
               gsem0, gsem1, gsem2, ssem0, ssem1, ssem2):
  c = lax.axis_index("c")
  s = lax.axis_index("s")
  gsems = (gsem0, gsem1, gsem2)
  ssems = (ssem0, ssem1, ssem2)

  zbuf = rows.at[0]

  def zero_body(i, _):
    rows[0, i, :] = jnp.zeros((F,), jnp.float32)
    return _
  lax.fori_loop(0, SCHUNK, zero_body, None)

  base = s * SLICE
  nfull = SLICE // SCHUNK
  rem = SLICE - nfull * SCHUNK
  for k in range(nfull):
    pltpu.sync_copy(zbuf, acc_sh.at[pl.ds(base + k * SCHUNK, SCHUNK)])
  if rem:
    pltpu.sync_copy(zbuf.at[pl.ds(0, rem)],
                    acc_sh.at[pl.ds(base + nfull * SCHUNK, rem)])
  plsc.subcore_barrier()

  def fire_gathers(b, g):
    rb = _worker_chunk_base(c, s, g, SJ)
    pltpu.sync_copy(src_hbm.at[pl.ds(rb, SJ)], sidx.at[b])
    pltpu.sync_copy(dst_hbm.at[pl.ds(rb, SJ)], didx.at[b])
    for j in range(SJ):
      pltpu.async_copy(y_hbm.at[sidx.at[b].at[j]],
                       rows.at[b].at[pl.ds(j * LANE, LANE)], gsems[b])

  def wait_gathers(b):
    for j in range(SJ):
      pltpu.make_async_copy(y_hbm.at[sidx.at[b].at[j]],
                            rows.at[b].at[pl.ds(j * LANE, LANE)],
                            gsems[b]).wait()

  def fire_scatters(b):
    for j in range(SJ):
      pltpu.async_copy(rows.at[b].at[pl.ds(j * LANE, LANE)],
                       acc_sh.at[didx.at[b].at[j]], ssems[b], add=True)

  def wait_scatters(b):
    for j in range(SJ):
      pltpu.make_async_copy(rows.at[b].at[pl.ds(j * LANE, LANE)],
                            acc_sh.at[didx.at[b].at[j]], ssems[b]).wait()

  fire_gathers(0, 0)
  fire_gathers(1, 1)

  def ring_body(i, _):
    for bb in range(3):
      g = 3 * i + 2 + bb
      b = (2 + bb) % 3

      if bb == 0:
        @pl.when(i > 0)
        def _drain():
          wait_scatters(b)
      else:
        wait_scatters(b)

      @pl.when(g < SCH)
      def _fire():
        fire_gathers(b, g)

      wait_gathers(bb)
      fire_scatters(bb)
    return _
  lax.fori_loop(0, SCH // 3, ring_body, None)

  wait_scatters((SCH - 1) % 3)

  plsc.subcore_barrier()
  for k in range(nfull):
    pltpu.sync_copy(acc_sh.at[pl.ds(base + k * SCHUNK, SCHUNK)], zbuf)
    pltpu.sync_copy(zbuf, acc_out.at[c, pl.ds(base + k * SCHUNK, SCHUNK)])
  if rem:
    pltpu.sync_copy(acc_sh.at[pl.ds(base + nfull * SCHUNK, rem)],
                    zbuf.at[pl.ds(0, rem)])
    pltpu.sync_copy(zbuf.at[pl.ds(0, rem)],
                    acc_out.at[c, pl.ds(base + nfull * SCHUNK, rem)])


R = 3584
GRID = NP // R


def _row_spec(cols):
  return pl.BlockSpec((R, cols), lambda i: (i, 0))


def _full_spec(shape):
  nd = len(shape)
  return pl.BlockSpec(shape, lambda i: (0,) * nd)


def _tc_stage1_body(x_ref, deg_ref, w1_ref, l1w_ref, l1b_ref,
                    l2w_ref, l2b_ref, dinv_ref, y1_ref, gg2_ref):
  d = deg_ref[...]
  deg_row = d[0:1, :] + d[1:2, :] + 1.0
  dinv = jnp.transpose(lax.rsqrt(deg_row)) * jnp.ones((1, F), jnp.float32)
  dinv_ref[...] = dinv
  x = x_ref[...]
  xw = jnp.dot(x, w1_ref[...], preferred_element_type=jnp.float32)
  y1_ref[...] = xw * dinv
  g1 = jnp.maximum(
      jnp.dot(x, l1w_ref[...], preferred_element_type=jnp.float32)
      + l1b_ref[...], 0.0)
  gg2_ref[...] = jnp.maximum(
      jnp.dot(g1, l2w_ref[...], preferred_element_type=jnp.float32)
      + l2b_ref[...], 0.0)


def _tc_stage2_body(acc_ref, y1_ref, dinv_ref, b1_ref, w2_ref,
                    a1w_ref, a1b_ref, a2w_ref, a2b_ref, y2_ref, xa1_ref):
  dinv = dinv_ref[...]
  x1 = jnp.maximum(
      dinv * (acc_ref[0] + acc_ref[1] + y1_ref[...]) + b1_ref[...], 0.0)
  y2_ref[...] = jnp.dot(
      x1, w2_ref[...], preferred_element_type=jnp.float32) * dinv
  t = jnp.maximum(
      jnp.dot(x1, a1w_ref[...], preferred_element_type=jnp.float32)
      + a1b_ref[...], 0.0)
  xa1_ref[...] = jnp.maximum(
      jnp.dot(t, a2w_ref[...], preferred_element_type=jnp.float32)
      + a2b_ref[...], 0.0)


def _tc_stage3_body(acc_ref, y2_ref, dinv_ref, b2_ref, gg2_ref,
                    xa1_ref, a3w_ref, a3b_ref, a4w_ref, a4b_ref,
                    f1a_ref, f1b_w_ref, f1c_ref, f1b_ref,
                    f2w_ref, f2b_ref, f3w_ref, f3b_ref, out_ref):
  dinv = dinv_ref[...]
  x2 = jnp.maximum(
      dinv * (acc_ref[0] + acc_ref[1] + y2_ref[...]) + b2_ref[...], 0.0)
  t = jnp.maximum(
      jnp.dot(x2, a3w_ref[...], preferred_element_type=jnp.float32)
      + a3b_ref[...], 0.0)
  xa2 = jnp.maximum(
      jnp.dot(t, a4w_ref[...], preferred_element_type=jnp.float32)
      + a4b_ref[...], 0.0)
  f = (jnp.dot(gg2_ref[...], f1a_ref[...], preferred_element_type=jnp.float32)
       + jnp.dot(xa1_ref[...], f1b_w_ref[...],
                 preferred_element_type=jnp.float32)
       + jnp.dot(xa2, f1c_ref[...], preferred_element_type=jnp.float32)
       + f1b_ref[...])
  f = jnp.maximum(f, 0.0)
  f = jnp.maximum(
      jnp.dot(f, f2w_ref[...], preferred_element_type=jnp.float32)
      + f2b_ref[...], 0.0)
  o = jnp.dot(f, f3w_ref[...], preferred_element_type=jnp.float32) + f3b_ref[...]
  out_ref[...] = jax.nn.sigmoid(o)


def kernel(x, edge_index, conv1_W, conv1_b, conv2_W, conv2_b,
           ln1_W, ln1_b, ln2_W, ln2_b,
           a1_W, a1_b, a2_W, a2_b, a3_W, a3_b, a4_W, a4_b,
           f1_W, f1_b, f2_W, f2_b, f3_W, f3_b):
  pad = E_PAD - E
  pad_idx = N + (jnp.arange(pad, dtype=jnp.int32) % 64)
  src2d = jnp.concatenate([edge_index[0], pad_idx]).reshape(IDX_ROWS, LANE)
  dst2d = jnp.concatenate([edge_index[1], pad_idx]).reshape(IDX_ROWS, LANE)

  xp = jnp.concatenate([x, jnp.zeros((NP - N, F), jnp.float32)])
  degp = _sc_degree(dst2d).reshape(NC, NP)

  deg_spec = pl.BlockSpec((NC, R), lambda i: (0, i))
  acc_spec = pl.BlockSpec((NC, R, F), lambda i: (0, i, 0))

  tc1 = pl.pallas_call(
      _tc_stage1_body,
      grid=(GRID,),
      in_specs=[
          _row_spec(F), deg_spec,
          _full_spec((F, F)), _full_spec((16, 32)), _full_spec((1, 32)),
          _full_spec((32, 16)), _full_spec((1, 16)),
      ],
      out_specs=[_row_spec(F), _row_spec(F), _row_spec(F)],
      out_shape=[
          jax.ShapeDtypeStruct((NP, F), jnp.float32),
          jax.ShapeDtypeStruct((NP, F), jnp.float32),
          jax.ShapeDtypeStruct((NP, F), jnp.float32),
      ],
  )
  dinv, y1, gg2 = tc1(xp, degp, conv1_W, ln1_W, ln1_b.reshape(1, 32),
                      ln2_W, ln2_b.reshape(1, 16))

  acc1 = _sc_segsum(y1, src2d, dst2d)

  tc2 = pl.pallas_call(
      _tc_stage2_body,
      grid=(GRID,),
      in_specs=[
          acc_spec, _row_spec(F), _row_spec(F),
          _full_spec((1, F)), _full_spec((F, F)),
          _full_spec((F, 16)), _full_spec((1, 16)),
          _full_spec((16, 16)), _full_spec((1, 16)),
      ],
      out_specs=[_row_spec(F), _row_spec(F)],
      out_shape=[
          jax.ShapeDtypeStruct((NP, F), jnp.float32),
          jax.ShapeDtypeStruct((NP, F), jnp.float32),
      ],
  )
  y2, xa1 = tc2(acc1, y1, dinv, conv1_b.reshape(1, F),
                conv2_W, a1_W, a1_b.reshape(1, 16), a2_W, a2_b.reshape(1, 16))

  acc2 = _sc_segsum(y2, src2d, dst2d)

  tc3 = pl.pallas_call(
      _tc_stage3_body,
      grid=(GRID,),
      in_specs=[
          acc_spec, _row_spec(F), _row_spec(F),
          _full_spec((1, F)), _row_spec(F), _row_spec(F),
          _full_spec((F, 16)), _full_spec((1, 16)),
          _full_spec((16, 16)), _full_spec((1, 16)),
          _full_spec((16, 64)), _full_spec((16, 64)), _full_spec((16, 64)),
          _full_spec((1, 64)),
          _full_spec((64, 32)), _full_spec((1, 32)),
          _full_spec((32, 1)), _full_spec((1, 1)),
      ],
      out_specs=[_row_spec(1)],
      out_shape=[jax.ShapeDtypeStruct((NP, 1), jnp.float32)],
  )
  (out,) = tc3(acc2, y2, dinv, conv2_b.reshape(1, F),
               gg2, xa1, a3_W, a3_b.reshape(1, 16), a4_W, a4_b.reshape(1, 16),
               f1_W[:16], f1_W[16:32], f1_W[32:48], f1_b.reshape(1, 64),
               f2_W, f2_b.reshape(1, 32), f3_W, f3_b.reshape(1, 1))
  return out[:N]


# speedup vs baseline: 78.4100x; 1.1969x over previous
import functools

import jax
import jax.numpy as jnp
from jax import lax
from jax.experimental import pallas as pl
from jax.experimental.pallas import tpu as pltpu
from jax.experimental.pallas import tpu_sc as plsc

N = 100000
F = 16
E = 3200000

NC = 2
NS = 16
NW = NC * NS

LANE = 128
ROWS_PER_W = 792
IDX_ROWS = NW * ROWS_PER_W
E_PAD = IDX_ROWS * LANE

DJ = 12
DCH = ROWS_PER_W // DJ

SJ = 4
SCHUNK = SJ * LANE
SCH = ROWS_PER_W // SJ

NP = 100352
SLICE = NP // NS
ZR = 1024

_sc_mesh = plsc.VectorSubcoreMesh(
    core_axis_name="c", subcore_axis_name="s", num_cores=NC, num_subcores=NS)


def _worker_chunk_base(c, s, g, jrows):
  wid = c * NS + s
  return wid * ROWS_PER_W + g * jrows


@functools.partial(
    pl.kernel,
    out_type=jax.ShapeDtypeStruct((NC * NP,), jnp.float32),
    mesh=_sc_mesh,
    compiler_params=pltpu.CompilerParams(use_tc_tiling_on_sc=False),
    scratch_types=[
        pltpu.VMEM_SHARED((NP,), jnp.float32),
        pltpu.VMEM((2, DJ, LANE), jnp.int32),
        pltpu.VMEM((LANE,), jnp.float32),
        pltpu.VMEM((SLICE,), jnp.float32),
        pltpu.SemaphoreType.DMA,
        pltpu.SemaphoreType.DMA,
    ],
)
def _sc_degree(dst_hbm, deg_out, deg_sh, didx, ones_v, bounce, sem0, sem1):
  c = lax.axis_index("c")
  s = lax.axis_index("s")
  sems = (sem0, sem1)

  for i in range(LANE // 16):
    ones_v[pl.ds(i * 16, 16)] = jnp.ones((16,), jnp.float32)

  def zero_body(i, _):
    bounce[pl.ds(i * 16, 16)] = jnp.zeros((16,), jnp.float32)
    return _
  lax.fori_loop(0, SLICE // 16, zero_body, None)
  pltpu.sync_copy(bounce, deg_sh.at[pl.ds(s * SLICE, SLICE)])
  plsc.subcore_barrier()

  def chunk_pair(i, _):
    for b in range(2):
      g = 2 * i + b

      @pl.when(i > 0)
      def _drain():
        for j in range(DJ):
          pltpu.make_async_copy(
              ones_v, deg_sh.at[didx.at[b].at[j]], sems[b]).wait()

      rb = _worker_chunk_base(c, s, g, DJ)
      pltpu.sync_copy(dst_hbm.at[pl.ds(rb, DJ)], didx.at[b])
      for j in range(DJ):
        pltpu.async_copy(ones_v, deg_sh.at[didx.at[b].at[j]], sems[b],
                         add=True)
    return _
  lax.fori_loop(0, DCH // 2, chunk_pair, None)
  for b in range(2):
    for j in range(DJ):
      pltpu.make_async_copy(
          ones_v, deg_sh.at[didx.at[b].at[j]], sems[b]).wait()

  plsc.subcore_barrier()
  pltpu.sync_copy(deg_sh.at[pl.ds(s * SLICE, SLICE)], bounce)
  pltpu.sync_copy(bounce, deg_out.at[pl.ds(c * NP + s * SLICE, SLICE)])


@functools.partial(
    pl.kernel,
    out_type=jax.ShapeDtypeStruct((NC, NP, F), jnp.float32),
    mesh=_sc_mesh,
    compiler_params=pltpu.CompilerParams(use_tc_tiling_on_sc=False),
    scratch_types=[
        pltpu.VMEM_SHARED((NP, F), jnp.float32),
        pltpu.VMEM((3, 2 * SJ, LANE), jnp.int32),
        pltpu.VMEM((3, SCHUNK, F), jnp.float32),
        pltpu.SemaphoreType.DMA,
        pltpu.SemaphoreType.DMA,
        pltpu.SemaphoreType.DMA,
        pltpu.SemaphoreType.DMA,
        pltpu.SemaphoreType.DMA,
        pltpu.SemaphoreType.DMA,
    ],
)
def _sc_segsum(y_hbm, eidx_hbm, acc_out,
               acc_sh, eidx, rows,
               gsem0, gsem1, gsem2, ssem0, ssem1, ssem2):
  c = lax.axis_index("c")
  s = lax.axis_index("s")
  gsems = (gsem0, gsem1, gsem2)
  ssems = (ssem0, ssem1, ssem2)

  zbuf = rows.at[0]

  def zero_body(i, _):
    rows[0, i, :] = jnp.zeros((F,), jnp.float32)
    return _
  lax.fori_loop(0, SCHUNK, zero_body, None)

  base = s * SLICE
  nfull = SLICE // SCHUNK
  rem = SLICE - nfull * SCHUNK
  for k in range(nfull):
    pltpu.sync_copy(zbuf, acc_sh.at[pl.ds(base + k * SCHUNK, SCHUNK)])
  if rem:
    pltpu.sync_copy(zbuf.at[pl.ds(0, rem)],
                    acc_sh.at[pl.ds(base + nfull * SCHUNK, rem)])
  plsc.subcore_barrier()

  def fire_gathers(b, g):
    rb = 2 * _worker_chunk_base(c, s, g, SJ)
    pltpu.sync_copy(eidx_hbm.at[pl.ds(rb, 2 * SJ)], eidx.at[b])
    for j in range(SJ):
      pltpu.async_copy(y_hbm.at[eidx.at[b].at[2 * j]],
                       rows.at[b].at[pl.ds(j * LANE, LANE)], gsems[b])

  def wait_gathers(b):
    for j in range(SJ):
      pltpu.make_async_copy(y_hbm.at[eidx.at[b].at[2 * j]],
                            rows.at[b].at[pl.ds(j * LANE, LANE)],
                            gsems[b]).wait()

  def fire_scatters(b):
    for j in range(SJ):
      pltpu.async_copy(rows.at[b].at[pl.ds(j * LANE, LANE)],
                       acc_sh.at[eidx.at[b].at[2 * j + 1]], ssems[b], add=True)

  def wait_scatters(b):
    for j in range(SJ):
      pltpu.make_async_copy(rows.at[b].at[pl.ds(j * LANE, LANE)],
                            acc_sh.at[eidx.at[b].at[2 * j + 1]], ssems[b]).wait()

  fire_gathers(0, 0)
  fire_gathers(1, 1)

  def ring_body(i, _):
    for bb in range(3):
      g = 3 * i + 2 + bb
      b = (2 + bb) % 3

      if bb == 0:
        @pl.when(i > 0)
        def _drain():
          wait_scatters(b)
      else:
        wait_scatters(b)

      @pl.when(g < SCH)
      def _fire():
        fire_gathers(b, g)

      wait_gathers(bb)
      fire_scatters(bb)
    return _
  lax.fori_loop(0, SCH // 3, ring_body, None)

  wait_scatters((SCH - 1) % 3)

  plsc.subcore_barrier()
  for k in range(nfull):
    pltpu.sync_copy(acc_sh.at[pl.ds(base + k * SCHUNK, SCHUNK)], zbuf)
    pltpu.sync_copy(zbuf, acc_out.at[c, pl.ds(base + k * SCHUNK, SCHUNK)])
  if rem:
    pltpu.sync_copy(acc_sh.at[pl.ds(base + nfull * SCHUNK, rem)],
                    zbuf.at[pl.ds(0, rem)])
    pltpu.sync_copy(zbuf.at[pl.ds(0, rem)],
                    acc_out.at[c, pl.ds(base + nfull * SCHUNK, rem)])


R = 3584
GRID = NP // R


def _row_spec(cols):
  return pl.BlockSpec((R, cols), lambda i: (i, 0))


def _full_spec(shape):
  nd = len(shape)
  return pl.BlockSpec(shape, lambda i: (0,) * nd)


def _tc_stage1_body(x_ref, deg_ref, w1_ref, l1w_ref, l1b_ref,
                    l2w_ref, l2b_ref, dinv_ref, y1_ref, gg2_ref):
  d = deg_ref[...]
  deg_row = d[0:1, :] + d[1:2, :] + 1.0
  dinv = jnp.transpose(lax.rsqrt(deg_row)) * jnp.ones((1, F), jnp.float32)
  dinv_ref[...] = dinv
  x = x_ref[...]
  xw = jnp.dot(x, w1_ref[...], preferred_element_type=jnp.float32)
  y1_ref[...] = xw * dinv
  g1 = jnp.maximum(
      jnp.dot(x, l1w_ref[...], preferred_element_type=jnp.float32)
      + l1b_ref[...], 0.0)
  gg2_ref[...] = jnp.maximum(
      jnp.dot(g1, l2w_ref[...], preferred_element_type=jnp.float32)
      + l2b_ref[...], 0.0)


def _tc_stage2_body(acc_ref, y1_ref, dinv_ref, b1_ref, w2_ref,
                    a1w_ref, a1b_ref, a2w_ref, a2b_ref, y2_ref, xa1_ref):
  dinv = dinv_ref[...]
  x1 = jnp.maximum(
      dinv * (acc_ref[0] + acc_ref[1] + y1_ref[...]) + b1_ref[...], 0.0)
  y2_ref[...] = jnp.dot(
      x1, w2_ref[...], preferred_element_type=jnp.float32) * dinv
  t = jnp.maximum(
      jnp.dot(x1, a1w_ref[...], preferred_element_type=jnp.float32)
      + a1b_ref[...], 0.0)
  xa1_ref[...] = jnp.maximum(
      jnp.dot(t, a2w_ref[...], preferred_element_type=jnp.float32)
      + a2b_ref[...], 0.0)


def _tc_stage3_body(acc_ref, y2_ref, dinv_ref, b2_ref, gg2_ref,
                    xa1_ref, a3w_ref, a3b_ref, a4w_ref, a4b_ref,
                    f1a_ref, f1b_w_ref, f1c_ref, f1b_ref,
                    f2w_ref, f2b_ref, f3w_ref, f3b_ref, out_ref):
  dinv = dinv_ref[...]
  x2 = jnp.maximum(
      dinv * (acc_ref[0] + acc_ref[1] + y2_ref[...]) + b2_ref[...], 0.0)
  t = jnp.maximum(
      jnp.dot(x2, a3w_ref[...], preferred_element_type=jnp.float32)
      + a3b_ref[...], 0.0)
  xa2 = jnp.maximum(
      jnp.dot(t, a4w_ref[...], preferred_element_type=jnp.float32)
      + a4b_ref[...], 0.0)
  f = (jnp.dot(gg2_ref[...], f1a_ref[...], preferred_element_type=jnp.float32)
       + jnp.dot(xa1_ref[...], f1b_w_ref[...],
                 preferred_element_type=jnp.float32)
       + jnp.dot(xa2, f1c_ref[...], preferred_element_type=jnp.float32)
       + f1b_ref[...])
  f = jnp.maximum(f, 0.0)
  f = jnp.maximum(
      jnp.dot(f, f2w_ref[...], preferred_element_type=jnp.float32)
      + f2b_ref[...], 0.0)
  o = jnp.dot(f, f3w_ref[...], preferred_element_type=jnp.float32) + f3b_ref[...]
  out_ref[...] = jax.nn.sigmoid(o)


def kernel(x, edge_index, conv1_W, conv1_b, conv2_W, conv2_b,
           ln1_W, ln1_b, ln2_W, ln2_b,
           a1_W, a1_b, a2_W, a2_b, a3_W, a3_b, a4_W, a4_b,
           f1_W, f1_b, f2_W, f2_b, f3_W, f3_b):
  pad = E_PAD - E
  pad_idx = N + (jnp.arange(pad, dtype=jnp.int32) % 64)
  src2d = jnp.concatenate([edge_index[0], pad_idx]).reshape(IDX_ROWS, LANE)
  dst2d = jnp.concatenate([edge_index[1], pad_idx]).reshape(IDX_ROWS, LANE)
  eidx2d = jnp.stack([src2d, dst2d], axis=1).reshape(2 * IDX_ROWS, LANE)

  xp = jnp.concatenate([x, jnp.zeros((NP - N, F), jnp.float32)])
  degp = _sc_degree(dst2d).reshape(NC, NP)

  deg_spec = pl.BlockSpec((NC, R), lambda i: (0, i))
  acc_spec = pl.BlockSpec((NC, R, F), lambda i: (0, i, 0))

  tc1 = pl.pallas_call(
      _tc_stage1_body,
      grid=(GRID,),
      in_specs=[
          _row_spec(F), deg_spec,
          _full_spec((F, F)), _full_spec((16, 32)), _full_spec((1, 32)),
          _full_spec((32, 16)), _full_spec((1, 16)),
      ],
      out_specs=[_row_spec(F), _row_spec(F), _row_spec(F)],
      out_shape=[
          jax.ShapeDtypeStruct((NP, F), jnp.float32),
          jax.ShapeDtypeStruct((NP, F), jnp.float32),
          jax.ShapeDtypeStruct((NP, F), jnp.float32),
      ],
  )
  dinv, y1, gg2 = tc1(xp, degp, conv1_W, ln1_W, ln1_b.reshape(1, 32),
                      ln2_W, ln2_b.reshape(1, 16))

  acc1 = _sc_segsum(y1, eidx2d)

  tc2 = pl.pallas_call(
      _tc_stage2_body,
      grid=(GRID,),
      in_specs=[
          acc_spec, _row_spec(F), _row_spec(F),
          _full_spec((1, F)), _full_spec((F, F)),
          _full_spec((F, 16)), _full_spec((1, 16)),
          _full_spec((16, 16)), _full_spec((1, 16)),
      ],
      out_specs=[_row_spec(F), _row_spec(F)],
      out_shape=[
          jax.ShapeDtypeStruct((NP, F), jnp.float32),
          jax.ShapeDtypeStruct((NP, F), jnp.float32),
      ],
  )
  y2, xa1 = tc2(acc1, y1, dinv, conv1_b.reshape(1, F),
                conv2_W, a1_W, a1_b.reshape(1, 16), a2_W, a2_b.reshape(1, 16))

  acc2 = _sc_segsum(y2, eidx2d)

  tc3 = pl.pallas_call(
      _tc_stage3_body,
      grid=(GRID,),
      in_specs=[
          acc_spec, _row_spec(F), _row_spec(F),
          _full_spec((1, F)), _row_spec(F), _row_spec(F),
          _full_spec((F, 16)), _full_spec((1, 16)),
          _full_spec((16, 16)), _full_spec((1, 16)),
          _full_spec((16, 64)), _full_spec((16, 64)), _full_spec((16, 64)),
          _full_spec((1, 64)),
          _full_spec((64, 32)), _full_spec((1, 32)),
          _full_spec((32, 1)), _full_spec((1, 1)),
      ],
      out_specs=[_row_spec(1)],
      out_shape=[jax.ShapeDtypeStruct((N, 1), jnp.float32)],
  )
  (out,) = tc3(acc2, y2, dinv, conv2_b.reshape(1, F),
               gg2, xa1, a3_W, a3_b.reshape(1, 16), a4_W, a4_b.reshape(1, 16),
               f1_W[:16], f1_W[16:32], f1_W[32:48], f1_b.reshape(1, 64),
               f2_W, f2_b.reshape(1, 32), f3_W, f3_b.reshape(1, 1))
  return out


# speedup vs baseline: 85.1552x; 1.0860x over previous
import functools

import jax
import jax.numpy as jnp
from jax import lax
from jax.experimental import pallas as pl
from jax.experimental.pallas import tpu as pltpu
from jax.experimental.pallas import tpu_sc as plsc

N = 100000
F = 16
E = 3200000

NC = 2
NS = 16
NW = NC * NS

LANE = 128
ROWS_PER_W = 792
IDX_ROWS = NW * ROWS_PER_W
E_PAD = IDX_ROWS * LANE

DJ = 12
DCH = ROWS_PER_W // DJ

SJ = 4
SCHUNK = SJ * LANE
SCH = ROWS_PER_W // SJ

NP = 100096
SLICE = NP // NS
ZR = 1024

_sc_mesh = plsc.VectorSubcoreMesh(
    core_axis_name="c", subcore_axis_name="s", num_cores=NC, num_subcores=NS)


def _worker_chunk_base(c, s, g, jrows):
  wid = c * NS + s
  return wid * ROWS_PER_W + g * jrows


@functools.partial(
    pl.kernel,
    out_type=jax.ShapeDtypeStruct((NC * NP,), jnp.float32),
    mesh=_sc_mesh,
    compiler_params=pltpu.CompilerParams(use_tc_tiling_on_sc=False),
    scratch_types=[
        pltpu.VMEM_SHARED((NP,), jnp.float32),
        pltpu.VMEM((2, DJ, LANE), jnp.int32),
        pltpu.VMEM((LANE,), jnp.float32),
        pltpu.VMEM((SLICE,), jnp.float32),
        pltpu.SemaphoreType.DMA,
        pltpu.SemaphoreType.DMA,
    ],
)
def _sc_degree(dst_hbm, deg_out, deg_sh, didx, ones_v, bounce, sem0, sem1):
  c = lax.axis_index("c")
  s = lax.axis_index("s")
  sems = (sem0, sem1)

  for i in range(LANE // 16):
    ones_v[pl.ds(i * 16, 16)] = jnp.ones((16,), jnp.float32)

  def zero_body(i, _):
    bounce[pl.ds(i * 16, 16)] = jnp.zeros((16,), jnp.float32)
    return _
  lax.fori_loop(0, SLICE // 16, zero_body, None)
  pltpu.sync_copy(bounce, deg_sh.at[pl.ds(s * SLICE, SLICE)])
  plsc.subcore_barrier()

  def chunk_pair(i, _):
    for b in range(2):
      g = 2 * i + b

      @pl.when(i > 0)
      def _drain():
        for j in range(DJ):
          pltpu.make_async_copy(
              ones_v, deg_sh.at[didx.at[b].at[j]], sems[b]).wait()

      rb = _worker_chunk_base(c, s, g, DJ)
      pltpu.sync_copy(dst_hbm.at[pl.ds(rb, DJ)], didx.at[b])
      for j in range(DJ):
        pltpu.async_copy(ones_v, deg_sh.at[didx.at[b].at[j]], sems[b],
                         add=True)
    return _
  lax.fori_loop(0, DCH // 2, chunk_pair, None)
  for b in range(2):
    for j in range(DJ):
      pltpu.make_async_copy(
          ones_v, deg_sh.at[didx.at[b].at[j]], sems[b]).wait()

  plsc.subcore_barrier()
  pltpu.sync_copy(deg_sh.at[pl.ds(s * SLICE, SLICE)], bounce)
  pltpu.sync_copy(bounce, deg_out.at[pl.ds(c * NP + s * SLICE, SLICE)])


@functools.partial(
    pl.kernel,
    out_type=jax.ShapeDtypeStruct((NC, NP, F), jnp.float32),
    mesh=_sc_mesh,
    compiler_params=pltpu.CompilerParams(use_tc_tiling_on_sc=False),
    scratch_types=[
        pltpu.VMEM_SHARED((NP, F), jnp.float32),
        pltpu.VMEM((6, 2 * SJ, LANE), jnp.int32),
        pltpu.VMEM((3, SCHUNK, F), jnp.float32),
        pltpu.SemaphoreType.DMA,
        pltpu.SemaphoreType.DMA,
        pltpu.SemaphoreType.DMA,
        pltpu.SemaphoreType.DMA,
        pltpu.SemaphoreType.DMA,
        pltpu.SemaphoreType.DMA,
        pltpu.SemaphoreType.DMA,
        pltpu.SemaphoreType.DMA,
        pltpu.SemaphoreType.DMA,
        pltpu.SemaphoreType.DMA,
        pltpu.SemaphoreType.DMA,
        pltpu.SemaphoreType.DMA,
    ],
)
def _sc_segsum(y_hbm, eidx_hbm, acc_out,
               acc_sh, eidx, rows,
               gsem0, gsem1, gsem2, ssem0, ssem1, ssem2,
               isem0, isem1, isem2, isem3, isem4, isem5):
  c = lax.axis_index("c")
  s = lax.axis_index("s")
  gsems = (gsem0, gsem1, gsem2)
  ssems = (ssem0, ssem1, ssem2)
  isems = (isem0, isem1, isem2, isem3, isem4, isem5)

  zbuf = rows.at[0]

  def zero_body(i, _):
    rows[0, i, :] = jnp.zeros((F,), jnp.float32)
    return _
  lax.fori_loop(0, SCHUNK, zero_body, None)

  base = s * SLICE
  nfull = SLICE // SCHUNK
  rem = SLICE - nfull * SCHUNK
  for k in range(nfull):
    pltpu.sync_copy(zbuf, acc_sh.at[pl.ds(base + k * SCHUNK, SCHUNK)])
  if rem:
    pltpu.sync_copy(zbuf.at[pl.ds(0, rem)],
                    acc_sh.at[pl.ds(base + nfull * SCHUNK, rem)])
  plsc.subcore_barrier()

  def idx_base(g):
    return 2 * _worker_chunk_base(c, s, g, SJ)

  def load_idx(bi, g):
    pltpu.async_copy(eidx_hbm.at[pl.ds(idx_base(g), 2 * SJ)], eidx.at[bi],
                     isems[bi])

  def wait_idx(bi, g):
    pltpu.make_async_copy(eidx_hbm.at[pl.ds(idx_base(g), 2 * SJ)],
                          eidx.at[bi], isems[bi]).wait()

  def fire_gathers(b, bi):
    for j in range(SJ):
      pltpu.async_copy(y_hbm.at[eidx.at[bi].at[2 * j]],
                       rows.at[b].at[pl.ds(j * LANE, LANE)], gsems[b])

  def wait_gathers(b, bi):
    for j in range(SJ):
      pltpu.make_async_copy(y_hbm.at[eidx.at[bi].at[2 * j]],
                            rows.at[b].at[pl.ds(j * LANE, LANE)],
                            gsems[b]).wait()

  def fire_scatters(b, bi):
    for j in range(SJ):
      pltpu.async_copy(rows.at[b].at[pl.ds(j * LANE, LANE)],
                       acc_sh.at[eidx.at[bi].at[2 * j + 1]], ssems[b],
                       add=True)

  def wait_scatters(b, bi):
    for j in range(SJ):
      pltpu.make_async_copy(rows.at[b].at[pl.ds(j * LANE, LANE)],
                            acc_sh.at[eidx.at[bi].at[2 * j + 1]],
                            ssems[b]).wait()

  load_idx(0, 0)
  load_idx(1, 1)
  wait_idx(0, 0)
  fire_gathers(0, 0)
  load_idx(2, 2)
  wait_idx(1, 1)
  fire_gathers(1, 1)
  load_idx(3, 3)

  def ring_body(i, _):
    for bb in range(6):
      g = 6 * i + 2 + bb
      b = (2 + bb) % 3
      bi = (2 + bb) % 6

      if bb == 0:
        @pl.when(i > 0)
        def _drain():
          wait_scatters(b, (2 + bb + 3) % 6)
      else:
        wait_scatters(b, (2 + bb + 3) % 6)

      @pl.when(g < SCH)
      def _gather():
        wait_idx(bi, g)
        fire_gathers(b, bi)

      @pl.when(g + 2 < SCH)
      def _prefetch():
        load_idx((bi + 2) % 6, g + 2)

      wait_gathers(bb % 3, bb % 6)
      fire_scatters(bb % 3, bb % 6)
    return _
  lax.fori_loop(0, SCH // 6, ring_body, None)

  wait_scatters((SCH - 1) % 3, (SCH - 1) % 6)

  plsc.subcore_barrier()
  for k in range(nfull):
    pltpu.sync_copy(acc_sh.at[pl.ds(base + k * SCHUNK, SCHUNK)], zbuf)
    pltpu.sync_copy(zbuf, acc_out.at[c, pl.ds(base + k * SCHUNK, SCHUNK)])
  if rem:
    pltpu.sync_copy(acc_sh.at[pl.ds(base + nfull * SCHUNK, rem)],
                    zbuf.at[pl.ds(0, rem)])
    pltpu.sync_copy(zbuf.at[pl.ds(0, rem)],
                    acc_out.at[c, pl.ds(base + nfull * SCHUNK, rem)])


R = 2176
GRID = NP // R


def _row_spec(cols):
  return pl.BlockSpec((R, cols), lambda i: (i, 0))


def _full_spec(shape):
  nd = len(shape)
  return pl.BlockSpec(shape, lambda i: (0,) * nd)


def _tc_stage1_body(x_ref, deg_ref, w1_ref, l1w_ref, l1b_ref,
                    l2w_ref, l2b_ref, dinv_ref, y1_ref, gg2_ref):
  d = deg_ref[...]
  deg_row = d[0:1, :] + d[1:2, :] + 1.0
  dinv = jnp.transpose(lax.rsqrt(deg_row)) * jnp.ones((1, F), jnp.float32)
  dinv_ref[...] = dinv
  x = x_ref[...]
  xw = jnp.dot(x, w1_ref[...], preferred_element_type=jnp.float32)
  y1_ref[...] = xw * dinv
  g1 = jnp.maximum(
      jnp.dot(x, l1w_ref[...], preferred_element_type=jnp.float32)
      + l1b_ref[...], 0.0)
  gg2_ref[...] = jnp.maximum(
      jnp.dot(g1, l2w_ref[...], preferred_element_type=jnp.float32)
      + l2b_ref[...], 0.0)


def _tc_stage2_body(acc_ref, y1_ref, dinv_ref, b1_ref, w2_ref,
                    a1w_ref, a1b_ref, a2w_ref, a2b_ref, y2_ref, xa1_ref):
  dinv = dinv_ref[...]
  x1 = jnp.maximum(
      dinv * (acc_ref[0] + acc_ref[1] + y1_ref[...]) + b1_ref[...], 0.0)
  y2_ref[...] = jnp.dot(
      x1, w2_ref[...], preferred_element_type=jnp.float32) * dinv
  t = jnp.maximum(
      jnp.dot(x1, a1w_ref[...], preferred_element_type=jnp.float32)
      + a1b_ref[...], 0.0)
  xa1_ref[...] = jnp.maximum(
      jnp.dot(t, a2w_ref[...], preferred_element_type=jnp.float32)
      + a2b_ref[...], 0.0)


def _tc_stage3_body(acc_ref, y2_ref, dinv_ref, b2_ref, gg2_ref,
                    xa1_ref, a3w_ref, a3b_ref, a4w_ref, a4b_ref,
                    f1a_ref, f1b_w_ref, f1c_ref, f1b_ref,
                    f2w_ref, f2b_ref, f3w_ref, f3b_ref, out_ref):
  dinv = dinv_ref[...]
  x2 = jnp.maximum(
      dinv * (acc_ref[0] + acc_ref[1] + y2_ref[...]) + b2_ref[...], 0.0)
  t = jnp.maximum(
      jnp.dot(x2, a3w_ref[...], preferred_element_type=jnp.float32)
      + a3b_ref[...], 0.0)
  xa2 = jnp.maximum(
      jnp.dot(t, a4w_ref[...], preferred_element_type=jnp.float32)
      + a4b_ref[...], 0.0)
  f = (jnp.dot(gg2_ref[...], f1a_ref[...], preferred_element_type=jnp.float32)
       + jnp.dot(xa1_ref[...], f1b_w_ref[...],
                 preferred_element_type=jnp.float32)
       + jnp.dot(xa2, f1c_ref[...], preferred_element_type=jnp.float32)
       + f1b_ref[...])
  f = jnp.maximum(f, 0.0)
  f = jnp.maximum(
      jnp.dot(f, f2w_ref[...], preferred_element_type=jnp.float32)
      + f2b_ref[...], 0.0)
  o = jnp.dot(f, f3w_ref[...], preferred_element_type=jnp.float32) + f3b_ref[...]
  out_ref[...] = jax.nn.sigmoid(o)


def kernel(x, edge_index, conv1_W, conv1_b, conv2_W, conv2_b,
           ln1_W, ln1_b, ln2_W, ln2_b,
           a1_W, a1_b, a2_W, a2_b, a3_W, a3_b, a4_W, a4_b,
           f1_W, f1_b, f2_W, f2_b, f3_W, f3_b):
  pad = E_PAD - E
  pad_idx = N + (jnp.arange(pad, dtype=jnp.int32) % 64)
  src2d = jnp.concatenate([edge_index[0], pad_idx]).reshape(IDX_ROWS, LANE)
  dst2d = jnp.concatenate([edge_index[1], pad_idx]).reshape(IDX_ROWS, LANE)
  eidx2d = jnp.stack([src2d, dst2d], axis=1).reshape(2 * IDX_ROWS, LANE)

  xp = jnp.concatenate([x, jnp.zeros((NP - N, F), jnp.float32)])
  degp = _sc_degree(dst2d).reshape(NC, NP)

  deg_spec = pl.BlockSpec((NC, R), lambda i: (0, i))
  acc_spec = pl.BlockSpec((NC, R, F), lambda i: (0, i, 0))

  tc1 = pl.pallas_call(
      _tc_stage1_body,
      grid=(GRID,),
      in_specs=[
          _row_spec(F), deg_spec,
          _full_spec((F, F)), _full_spec((16, 32)), _full_spec((1, 32)),
          _full_spec((32, 16)), _full_spec((1, 16)),
      ],
      out_specs=[_row_spec(F), _row_spec(F), _row_spec(F)],
      out_shape=[
          jax.ShapeDtypeStruct((NP, F), jnp.float32),
          jax.ShapeDtypeStruct((NP, F), jnp.float32),
          jax.ShapeDtypeStruct((NP, F), jnp.float32),
      ],
  )
  dinv, y1, gg2 = tc1(xp, degp, conv1_W, ln1_W, ln1_b.reshape(1, 32),
                      ln2_W, ln2_b.reshape(1, 16))

  acc1 = _sc_segsum(y1, eidx2d)

  tc2 = pl.pallas_call(
      _tc_stage2_body,
      grid=(GRID,),
      in_specs=[
          acc_spec, _row_spec(F), _row_spec(F),
          _full_spec((1, F)), _full_spec((F, F)),
          _full_spec((F, 16)), _full_spec((1, 16)),
          _full_spec((16, 16)), _full_spec((1, 16)),
      ],
      out_specs=[_row_spec(F), _row_spec(F)],
      out_shape=[
          jax.ShapeDtypeStruct((NP, F), jnp.float32),
          jax.ShapeDtypeStruct((NP, F), jnp.float32),
      ],
  )
  y2, xa1 = tc2(acc1, y1, dinv, conv1_b.reshape(1, F),
                conv2_W, a1_W, a1_b.reshape(1, 16), a2_W, a2_b.reshape(1, 16))

  acc2 = _sc_segsum(y2, eidx2d)

  tc3 = pl.pallas_call(
      _tc_stage3_body,
      grid=(GRID,),
      in_specs=[
          acc_spec, _row_spec(F), _row_spec(F),
          _full_spec((1, F)), _row_spec(F), _row_spec(F),
          _full_spec((F, 16)), _full_spec((1, 16)),
          _full_spec((16, 16)), _full_spec((1, 16)),
          _full_spec((16, 64)), _full_spec((16, 64)), _full_spec((16, 64)),
          _full_spec((1, 64)),
          _full_spec((64, 32)), _full_spec((1, 32)),
          _full_spec((32, 1)), _full_spec((1, 1)),
      ],
      out_specs=[_row_spec(1)],
      out_shape=[jax.ShapeDtypeStruct((N, 1), jnp.float32)],
  )
  (out,) = tc3(acc2, y2, dinv, conv2_b.reshape(1, F),
               gg2, xa1, a3_W, a3_b.reshape(1, 16), a4_W, a4_b.reshape(1, 16),
               f1_W[:16], f1_W[16:32], f1_W[32:48], f1_b.reshape(1, 64),
               f2_W, f2_b.reshape(1, 32), f3_W, f3_b.reshape(1, 1))
  return out


# speedup vs baseline: 129.6116x; 1.5221x over previous
import functools

import jax
import jax.numpy as jnp
from jax import lax
from jax.experimental import pallas as pl
from jax.experimental.pallas import tpu as pltpu
from jax.experimental.pallas import tpu_sc as plsc

N = 100000
F = 16
E = 3200000

NC = 2
NS = 16
NW = NC * NS

LANE = 128
ROWS_PER_W = 792
IDX_ROWS = NW * ROWS_PER_W
E_PAD = IDX_ROWS * LANE

DJ = 12
DCH = ROWS_PER_W // DJ

SJ = 4
SCHUNK = SJ * LANE
SCH = ROWS_PER_W // SJ

NP = 100096
SLICE = NP // NS
ZR = 1024

_sc_mesh = plsc.VectorSubcoreMesh(
    core_axis_name="c", subcore_axis_name="s", num_cores=NC, num_subcores=NS)


def _worker_chunk_base(c, s, g, jrows):
  wid = c * NS + s
  return wid * ROWS_PER_W + g * jrows


@functools.partial(
    pl.kernel,
    out_type=jax.ShapeDtypeStruct((NC, NP, F), jnp.float32),
    mesh=_sc_mesh,
    compiler_params=pltpu.CompilerParams(
        use_tc_tiling_on_sc=False, needs_layout_passes=False),
    scratch_types=[
        pltpu.VMEM_SHARED((NP,), jnp.float32),
        pltpu.VMEM((2, DJ, LANE), jnp.int32),
        pltpu.VMEM((LANE,), jnp.float32),
        pltpu.VMEM((SLICE,), jnp.float32),
        pltpu.VMEM((512, F), jnp.float32),
        pltpu.SemaphoreType.DMA,
        pltpu.SemaphoreType.DMA,
    ],
)
def _sc_degree(dst_hbm, deg_out, deg_sh, didx, ones_v, bounce, b16,
               sem0, sem1):
  c = lax.axis_index("c")
  s = lax.axis_index("s")
  sems = (sem0, sem1)

  for i in range(LANE // 16):
    ones_v[pl.ds(i * 16, 16)] = jnp.ones((16,), jnp.float32)

  def zero_body(i, _):
    bounce[pl.ds(i * 16, 16)] = jnp.zeros((16,), jnp.float32)
    return _
  lax.fori_loop(0, SLICE // 16, zero_body, None)
  pltpu.sync_copy(bounce, deg_sh.at[pl.ds(s * SLICE, SLICE)])
  plsc.subcore_barrier()

  def chunk_pair(i, _):
    for b in range(2):
      g = 2 * i + b

      @pl.when(i > 0)
      def _drain():
        for j in range(DJ):
          pltpu.make_async_copy(
              ones_v, deg_sh.at[didx.at[b].at[j]], sems[b]).wait()

      rb = _worker_chunk_base(c, s, g, DJ)
      pltpu.sync_copy(dst_hbm.at[pl.ds(rb, DJ)], didx.at[b])
      for j in range(DJ):
        pltpu.async_copy(ones_v, deg_sh.at[didx.at[b].at[j]], sems[b],
                         add=True)
    return _
  lax.fori_loop(0, DCH // 2, chunk_pair, None)
  for b in range(2):
    for j in range(DJ):
      pltpu.make_async_copy(
          ones_v, deg_sh.at[didx.at[b].at[j]], sems[b]).wait()

  plsc.subcore_barrier()
  pltpu.sync_copy(deg_sh.at[pl.ds(s * SLICE, SLICE)], bounce)
  nfull = SLICE // 512
  rem = SLICE - nfull * 512
  for k in range(nfull + 1):
    cnt = 512 if k < nfull else rem
    if cnt == 0:
      break

    def rep_body(i, _):
      b16[i, :] = plsc.load_gather(
          bounce, [jnp.full((16,), k * 512 + i, jnp.int32)])
      return _
    lax.fori_loop(0, cnt, rep_body, None)
    pltpu.sync_copy(
        b16.at[pl.ds(0, cnt)],
        deg_out.at[c, pl.ds(s * SLICE + k * 512, cnt)])


@functools.partial(
    pl.kernel,
    out_type=jax.ShapeDtypeStruct((NC, NP, F), jnp.float32),
    mesh=_sc_mesh,
    compiler_params=pltpu.CompilerParams(use_tc_tiling_on_sc=False),
    scratch_types=[
        pltpu.VMEM_SHARED((NP, F), jnp.float32),
        pltpu.VMEM((6, 2 * SJ, LANE), jnp.int32),
        pltpu.VMEM((3, SCHUNK, F), jnp.float32),
        pltpu.SemaphoreType.DMA,
        pltpu.SemaphoreType.DMA,
        pltpu.SemaphoreType.DMA,
        pltpu.SemaphoreType.DMA,
        pltpu.SemaphoreType.DMA,
        pltpu.SemaphoreType.DMA,
        pltpu.SemaphoreType.DMA,
        pltpu.SemaphoreType.DMA,
        pltpu.SemaphoreType.DMA,
        pltpu.SemaphoreType.DMA,
        pltpu.SemaphoreType.DMA,
        pltpu.SemaphoreType.DMA,
    ],
)
def _sc_segsum(y_hbm, eidx_hbm, acc_out,
               acc_sh, eidx, rows,
               gsem0, gsem1, gsem2, ssem0, ssem1, ssem2,
               isem0, isem1, isem2, isem3, isem4, isem5):
  c = lax.axis_index("c")
  s = lax.axis_index("s")
  gsems = (gsem0, gsem1, gsem2)
  ssems = (ssem0, ssem1, ssem2)
  isems = (isem0, isem1, isem2, isem3, isem4, isem5)

  zbuf = rows.at[0]

  def zero_body(i, _):
    rows[0, i, :] = jnp.zeros((F,), jnp.float32)
    return _
  lax.fori_loop(0, SCHUNK, zero_body, None)

  base = s * SLICE
  nfull = SLICE // SCHUNK
  rem = SLICE - nfull * SCHUNK
  for k in range(nfull):
    pltpu.sync_copy(zbuf, acc_sh.at[pl.ds(base + k * SCHUNK, SCHUNK)])
  if rem:
    pltpu.sync_copy(zbuf.at[pl.ds(0, rem)],
                    acc_sh.at[pl.ds(base + nfull * SCHUNK, rem)])
  plsc.subcore_barrier()

  def idx_base(g):
    return 2 * _worker_chunk_base(c, s, g, SJ)

  def load_idx(bi, g):
    pltpu.async_copy(eidx_hbm.at[pl.ds(idx_base(g), 2 * SJ)], eidx.at[bi],
                     isems[bi])

  def wait_idx(bi, g):
    pltpu.make_async_copy(eidx_hbm.at[pl.ds(idx_base(g), 2 * SJ)],
                          eidx.at[bi], isems[bi]).wait()

  def fire_gathers(b, bi):
    for j in range(SJ):
      pltpu.async_copy(y_hbm.at[eidx.at[bi].at[2 * j]],
                       rows.at[b].at[pl.ds(j * LANE, LANE)], gsems[b])

  def wait_gathers(b, bi):
    for j in range(SJ):
      pltpu.make_async_copy(y_hbm.at[eidx.at[bi].at[2 * j]],
                            rows.at[b].at[pl.ds(j * LANE, LANE)],
                            gsems[b]).wait()

  def fire_scatters(b, bi):
    for j in range(SJ):
      pltpu.async_copy(rows.at[b].at[pl.ds(j * LANE, LANE)],
                       acc_sh.at[eidx.at[bi].at[2 * j + 1]], ssems[b],
                       add=True)

  def wait_scatters(b, bi):
    for j in range(SJ):
      pltpu.make_async_copy(rows.at[b].at[pl.ds(j * LANE, LANE)],
                            acc_sh.at[eidx.at[bi].at[2 * j + 1]],
                            ssems[b]).wait()

  load_idx(0, 0)
  load_idx(1, 1)
  wait_idx(0, 0)
  fire_gathers(0, 0)
  load_idx(2, 2)
  wait_idx(1, 1)
  fire_gathers(1, 1)
  load_idx(3, 3)

  def ring_body(i, _):
    for bb in range(6):
      g = 6 * i + 2 + bb
      b = (2 + bb) % 3
      bi = (2 + bb) % 6

      if bb == 0:
        @pl.when(i > 0)
        def _drain():
          wait_scatters(b, (2 + bb + 3) % 6)
      else:
        wait_scatters(b, (2 + bb + 3) % 6)

      @pl.when(g < SCH)
      def _gather():
        wait_idx(bi, g)
        fire_gathers(b, bi)

      @pl.when(g + 2 < SCH)
      def _prefetch():
        load_idx((bi + 2) % 6, g + 2)

      wait_gathers(bb % 3, bb % 6)
      fire_scatters(bb % 3, bb % 6)
    return _
  lax.fori_loop(0, SCH // 6, ring_body, None)

  wait_scatters((SCH - 1) % 3, (SCH - 1) % 6)

  plsc.subcore_barrier()
  for k in range(nfull):
    pltpu.sync_copy(acc_sh.at[pl.ds(base + k * SCHUNK, SCHUNK)], zbuf)
    pltpu.sync_copy(zbuf, acc_out.at[c, pl.ds(base + k * SCHUNK, SCHUNK)])
  if rem:
    pltpu.sync_copy(acc_sh.at[pl.ds(base + nfull * SCHUNK, rem)],
                    zbuf.at[pl.ds(0, rem)])
    pltpu.sync_copy(zbuf.at[pl.ds(0, rem)],
                    acc_out.at[c, pl.ds(base + nfull * SCHUNK, rem)])


R = 2176
GRID = NP // R
NPK = NP // 8
RK = R // 8


def _row_spec(cols):
  return pl.BlockSpec((RK, cols), lambda i: (i, 0))


def _full_spec(shape):
  nd = len(shape)
  return pl.BlockSpec(shape, lambda i: (0,) * nd)


def _tc_stage1_body(x_ref, deg_ref, w1_ref, l1w_ref, l1b_ref,
                    l2w_ref, l2b_ref, dinv_ref, y1_ref, gg2_ref):
  d = deg_ref[...]
  dinv = lax.rsqrt(d[0] + d[1] + 1.0)
  dinv_ref[...] = dinv
  x = x_ref[...]
  xw = jnp.dot(x, w1_ref[...], preferred_element_type=jnp.float32)
  y1_ref[...] = xw * dinv
  g1 = jnp.maximum(
      jnp.dot(x, l1w_ref[...], preferred_element_type=jnp.float32)
      + l1b_ref[...], 0.0)
  gg2_ref[...] = jnp.maximum(
      jnp.dot(g1, l2w_ref[...], preferred_element_type=jnp.float32)
      + l2b_ref[...], 0.0)


def _tc_stage2_body(acc_ref, y1_ref, dinv_ref, b1_ref, w2_ref,
                    a1w_ref, a1b_ref, a2w_ref, a2b_ref, y2_ref, xa1_ref):
  dinv = dinv_ref[...]
  x1 = jnp.maximum(
      dinv * (acc_ref[0] + acc_ref[1] + y1_ref[...]) + b1_ref[...], 0.0)
  y2_ref[...] = jnp.dot(
      x1, w2_ref[...], preferred_element_type=jnp.float32) * dinv
  t = jnp.maximum(
      jnp.dot(x1, a1w_ref[...], preferred_element_type=jnp.float32)
      + a1b_ref[...], 0.0)
  xa1_ref[...] = jnp.maximum(
      jnp.dot(t, a2w_ref[...], preferred_element_type=jnp.float32)
      + a2b_ref[...], 0.0)


def _tc_stage3_body(acc_ref, y2_ref, dinv_ref, b2_ref, gg2_ref,
                    xa1_ref, a3w_ref, a3b_ref, a4w_ref, a4b_ref,
                    f1a_ref, f1b_w_ref, f1c_ref, f1b_ref,
                    f2w_ref, f2b_ref, f3w_ref, f3b_ref, out_ref):
  dinv = dinv_ref[...]
  x2 = jnp.maximum(
      dinv * (acc_ref[0] + acc_ref[1] + y2_ref[...]) + b2_ref[...], 0.0)
  t = jnp.maximum(
      jnp.dot(x2, a3w_ref[...], preferred_element_type=jnp.float32)
      + a3b_ref[...], 0.0)
  xa2 = jnp.maximum(
      jnp.dot(t, a4w_ref[...], preferred_element_type=jnp.float32)
      + a4b_ref[...], 0.0)
  f = (jnp.dot(gg2_ref[...], f1a_ref[...], preferred_element_type=jnp.float32)
       + jnp.dot(xa1_ref[...], f1b_w_ref[...],
                 preferred_element_type=jnp.float32)
       + jnp.dot(xa2, f1c_ref[...], preferred_element_type=jnp.float32)
       + f1b_ref[...])
  f = jnp.maximum(f, 0.0)
  f = jnp.maximum(
      jnp.dot(f, f2w_ref[...], preferred_element_type=jnp.float32)
      + f2b_ref[...], 0.0)
  o = jnp.dot(f, f3w_ref[...], preferred_element_type=jnp.float32) + f3b_ref[...]
  out_ref[...] = jax.nn.sigmoid(o)


def kernel(x, edge_index, conv1_W, conv1_b, conv2_W, conv2_b,
           ln1_W, ln1_b, ln2_W, ln2_b,
           a1_W, a1_b, a2_W, a2_b, a3_W, a3_b, a4_W, a4_b,
           f1_W, f1_b, f2_W, f2_b, f3_W, f3_b):
  pad = E_PAD - E
  pad_idx = N + (jnp.arange(pad, dtype=jnp.int32) % 64)
  src2d = jnp.concatenate([edge_index[0], pad_idx]).reshape(IDX_ROWS, LANE)
  dst2d = jnp.concatenate([edge_index[1], pad_idx]).reshape(IDX_ROWS, LANE)
  eidx2d = jnp.stack([src2d, dst2d], axis=1).reshape(2 * IDX_ROWS, LANE)

  eye8 = jnp.eye(8, dtype=jnp.float32)

  def kr(w):
    return jnp.kron(eye8, w)

  def bt(b):
    return jnp.tile(b, 8)[None, :]

  xp8 = jnp.concatenate(
      [x, jnp.zeros((NP - N, F), jnp.float32)]).reshape(NPK, 128)
  deg16 = _sc_degree(dst2d).reshape(NC, NPK, 128)

  acc_spec = pl.BlockSpec((NC, RK, 128), lambda i: (0, i, 0))

  tc1 = pl.pallas_call(
      _tc_stage1_body,
      grid=(GRID,),
      in_specs=[
          _row_spec(128), acc_spec,
          _full_spec((128, 128)), _full_spec((128, 256)),
          _full_spec((1, 256)),
          _full_spec((256, 128)), _full_spec((1, 128)),
      ],
      out_specs=[_row_spec(128), _row_spec(128), _row_spec(128)],
      out_shape=[
          jax.ShapeDtypeStruct((NPK, 128), jnp.float32),
          jax.ShapeDtypeStruct((NPK, 128), jnp.float32),
          jax.ShapeDtypeStruct((NPK, 128), jnp.float32),
      ],
  )
  dinv, y1, gg2 = tc1(xp8, deg16, kr(conv1_W), kr(ln1_W), bt(ln1_b),
                      kr(ln2_W), bt(ln2_b))

  acc1 = _sc_segsum(y1.reshape(NP, F), eidx2d)

  tc2 = pl.pallas_call(
      _tc_stage2_body,
      grid=(GRID,),
      in_specs=[
          acc_spec, _row_spec(128), _row_spec(128),
          _full_spec((1, 128)), _full_spec((128, 128)),
          _full_spec((128, 128)), _full_spec((1, 128)),
          _full_spec((128, 128)), _full_spec((1, 128)),
      ],
      out_specs=[_row_spec(128), _row_spec(128)],
      out_shape=[
          jax.ShapeDtypeStruct((NPK, 128), jnp.float32),
          jax.ShapeDtypeStruct((NPK, 128), jnp.float32),
      ],
  )
  y2, xa1 = tc2(acc1.reshape(NC, NPK, 128), y1, dinv, bt(conv1_b),
                kr(conv2_W), kr(a1_W), bt(a1_b), kr(a2_W), bt(a2_b))

  acc2 = _sc_segsum(y2.reshape(NP, F), eidx2d)

  tc3 = pl.pallas_call(
      _tc_stage3_body,
      grid=(GRID,),
      in_specs=[
          acc_spec, _row_spec(128), _row_spec(128),
          _full_spec((1, 128)), _row_spec(128), _row_spec(128),
          _full_spec((128, 128)), _full_spec((1, 128)),
          _full_spec((128, 128)), _full_spec((1, 128)),
          _full_spec((128, 512)), _full_spec((128, 512)),
          _full_spec((128, 512)), _full_spec((1, 512)),
          _full_spec((512, 256)), _full_spec((1, 256)),
          _full_spec((256, 8)), _full_spec((1, 8)),
      ],
      out_specs=[_row_spec(8)],
      out_shape=[jax.ShapeDtypeStruct((NPK, 8), jnp.float32)],
  )
  (out8,) = tc3(acc2.reshape(NC, NPK, 128), y2, dinv, bt(conv2_b),
                gg2, xa1, kr(a3_W), bt(a3_b), kr(a4_W), bt(a4_b),
                kr(f1_W[:16]), kr(f1_W[16:32]), kr(f1_W[32:48]), bt(f1_b),
                kr(f2_W), bt(f2_b), kr(f3_W), bt(f3_b))
  return out8[:N // 8].reshape(N, 1)
